# baseline (speedup 1.0000x reference)
import dataclasses
import functools

import jax
import jax.numpy as jnp
import numpy as np
from jax import lax
from jax.experimental import pallas as pl
from jax.experimental.pallas import tpu as pltpu
from jax.experimental.pallas import tpu_sc as plsc

N = 16384
NNZ = 262144
B = 64

NUM_CORES = 2
NUM_SUBCORES = 16
LANES = 16

G = 128
GRP = 4
CRING = 4
GRING = 4
SRING = 2
EDGES_PER_TILE = NNZ // NUM_SUBCORES
BLOCKS = EDGES_PER_TILE // G
NGRP = BLOCKS // GRP
ROWS_PER_TILE = N // NUM_SUBCORES


def _transpose_body(x_ref, o_ref):
    x = x_ref[...]
    groups = []
    for h in range(B // 32):
        a = x[32 * h:32 * h + 16]
        b = x[32 * h + 16:32 * h + 32]
        groups.append(jnp.stack([a, b], axis=1).reshape(32, x.shape[1]))
    o_ref[...] = jnp.concatenate(groups, axis=0).T.astype(jnp.bfloat16)


def _features_to_table(features):
    blk = 2048
    return pl.pallas_call(
        _transpose_body,
        grid=(N // blk,),
        in_specs=[pl.BlockSpec((B, blk), lambda i: (0, i))],
        out_specs=pl.BlockSpec((blk, B), lambda i: (i, 0)),
        out_shape=jax.ShapeDtypeStruct((N, B), jnp.bfloat16),
    )(features)


def _combine_body(p_ref, k_ref, b_ref, o_ref):
    r = p_ref[0] * k_ref[0] + p_ref[1] * k_ref[1] + b_ref[0]
    o_ref[...] = jnp.clip(r, 0.0, 1.0).T


def _combine(partials, kern, bias):
    blk = 2048
    return pl.pallas_call(
        _combine_body,
        grid=(N // blk,),
        in_specs=[
            pl.BlockSpec((2, blk, B), lambda i: (0, i, 0)),
            pl.BlockSpec(memory_space=pltpu.SMEM),
            pl.BlockSpec(memory_space=pltpu.SMEM),
        ],
        out_specs=pl.BlockSpec((B, blk), lambda i: (0, i)),
        out_shape=jax.ShapeDtypeStruct((B, N), jnp.float32),
    )(partials, kern, bias)


def _sc_body(ft_hbm, cols_hbm, rows_hbm, vals_hbm, out_hbm,
             cols_c, rows_c, vals_c, gbuf, sbuf, acc,
             csems, gsems, ssems):
    c = lax.axis_index("c")
    s = lax.axis_index("s")

    def stage_slice(g):
        return pl.ds(c * (NNZ // G) + s * BLOCKS + g * GRP, GRP)

    def stage_start(g, q):
        sl = stage_slice(g)
        pltpu.async_copy(cols_hbm.at[sl], cols_c.at[q], csems[q])
        pltpu.async_copy(rows_hbm.at[sl], rows_c.at[q], csems[q])
        pltpu.async_copy(vals_hbm.at[sl], vals_c.at[q], csems[q])

    def stage_wait(g, q):
        sl = stage_slice(g)
        pltpu.make_async_copy(cols_hbm.at[sl], cols_c.at[q], csems[q]).wait()
        pltpu.make_async_copy(rows_hbm.at[sl], rows_c.at[q], csems[q]).wait()
        pltpu.make_async_copy(vals_hbm.at[sl], vals_c.at[q], csems[q]).wait()

    def gather(q, jj, gj):
        return pltpu.make_async_copy(ft_hbm.at[cols_c.at[q, jj]],
                                     gbuf.at[gj], gsems[gj])

    def scatter(q, jj, sj):
        return pltpu.make_async_copy(sbuf.at[sj], acc.at[rows_c.at[q, jj]],
                                     ssems[sj])

    def compute(q, jj, gj, sj):
        @plsc.parallel_loop(0, G, unroll=8)
        def _(e):
            val = plsc.load_gather(
                vals_c.at[q, jj], [jnp.full((LANES,), e, jnp.int32)])
            for h in range(B // (2 * LANES)):
                gv = gbuf[gj, e, pl.ds(2 * LANES * h, 2 * LANES)]
                a, b = plsc.unpack(gv, format=plsc.PackFormat.INTERLEAVED)
                sbuf[sj, e, pl.ds(2 * LANES * h, LANES)] = a * val
                sbuf[sj, e, pl.ds(2 * LANES * h + LANES, LANES)] = b * val

    for q in range(min(3, CRING - 1)):
        stage_start(q, q)

    zero = jnp.zeros((LANES,), jnp.float32)

    @pl.loop(0, G)
    def _(i):
        for k in range(B // LANES):
            sbuf[0, i, pl.ds(k * LANES, LANES)] = zero

    for j in range(ROWS_PER_TILE // G):
        pltpu.sync_copy(sbuf.at[0], acc.at[pl.ds(s * ROWS_PER_TILE + j * G, G)])

    plsc.subcore_barrier()

    stage_wait(0, 0)
    stage_wait(1, 1)
    for jj in range(GRING):
        gather(0, jj, jj).start()

    @pl.loop(0, NGRP // CRING)
    def _(m):
        for q in range(CRING):
            g = m * CRING + q
            for jj in range(GRP):
                bb = g * GRP + jj
                gj = jj % GRING
                sj = jj % SRING

                if jj == 0:
                    @pl.when(jnp.logical_and(g + 1 < NGRP, g > 0))
                    def _():
                        stage_wait(g + 1, (q + 1) % CRING)

                if jj == 2:
                    @pl.when(g + 3 < NGRP)
                    def _():
                        stage_start(g + 3, (q + 3) % CRING)

                gather(q, jj, gj).wait()

                if jj >= SRING:
                    scatter(q, jj - SRING, sj).wait()
                else:
                    @pl.when(g > 0)
                    def _():
                        scatter((q + CRING - 1) % CRING, GRP - SRING + jj,
                                sj).wait()

                compute(q, jj, gj, sj)

                @pl.when(bb + GRING < BLOCKS)
                def _():
                    gather((q + 1) % CRING, jj, gj).start()

                scatter(q, jj, sj).start(add=True)

    for u in range(SRING):
        scatter(CRING - 1, GRP - SRING + u, u % SRING).wait()

    plsc.subcore_barrier()

    for j in range(ROWS_PER_TILE // G):
        sl = pl.ds(s * ROWS_PER_TILE + j * G, G)
        pltpu.sync_copy(acc.at[sl], out_hbm.at[c].at[sl])


def _spmm_sc(ft, cols, rows, vals):
    mesh = plsc.VectorSubcoreMesh(
        core_axis_name="c", subcore_axis_name="s",
        num_cores=NUM_CORES, num_subcores=NUM_SUBCORES)
    cp = pltpu.CompilerParams()
    if "needs_layout_passes" in pltpu.CompilerParams.__dataclass_fields__:
        cp = dataclasses.replace(cp, needs_layout_passes=False)
    if "use_tc_tiling_on_sc" in pltpu.CompilerParams.__dataclass_fields__:
        cp = dataclasses.replace(cp, use_tc_tiling_on_sc=False)
    fn = pl.kernel(
        _sc_body,
        out_type=jax.ShapeDtypeStruct((2, N, B), jnp.float32),
        mesh=mesh,
        compiler_params=cp,
        scratch_types=[
            pltpu.VMEM((CRING, GRP, G), jnp.int32),
            pltpu.VMEM((CRING, GRP, G), jnp.int32),
            pltpu.VMEM((CRING, GRP, G), jnp.float32),
            pltpu.VMEM((GRING, G, B), jnp.bfloat16),
            pltpu.VMEM((SRING, G, B), jnp.float32),
            pltpu.VMEM_SHARED((N, B), jnp.float32),
            [pltpu.SemaphoreType.DMA] * CRING,
            [pltpu.SemaphoreType.DMA] * GRING,
            [pltpu.SemaphoreType.DMA] * SRING,
        ],
    )
    return fn(ft, cols, rows, vals)


def kernel(features, vals0, vals1, kernel, bias, rows0, cols0, rows1, cols1):
    x = features.reshape(2, 2, 16, N)
    x = jnp.transpose(x, (0, 2, 1, 3)).reshape(B, N)
    ft = x.T.astype(jnp.bfloat16)
    shp = (2 * NNZ // G, G)
    partials = _spmm_sc(ft,
                        jnp.concatenate([cols0, cols1]).reshape(shp),
                        jnp.concatenate([rows0, rows1]).reshape(shp),
                        jnp.concatenate([vals0, vals1]).reshape(shp))
    return _combine(partials, kernel.reshape(2), bias)


# speedup vs baseline: 1.0332x; 1.0332x over previous
import dataclasses
import functools

import jax
import jax.numpy as jnp
import numpy as np
from jax import lax
from jax.experimental import pallas as pl
from jax.experimental.pallas import tpu as pltpu
from jax.experimental.pallas import tpu_sc as plsc

N = 16384
NNZ = 262144
B = 64

NUM_CORES = 2
NUM_SUBCORES = 16
LANES = 16

G = 128
GRP = 4
CRING = 4
GRING = 4
SRING = 4
EDGES_PER_TILE = NNZ // NUM_SUBCORES
BLOCKS = EDGES_PER_TILE // G
NGRP = BLOCKS // GRP
ROWS_PER_TILE = N // NUM_SUBCORES


def _transpose_body(x_ref, o_ref):
    x = x_ref[...]
    groups = []
    for h in range(B // 32):
        a = x[32 * h:32 * h + 16]
        b = x[32 * h + 16:32 * h + 32]
        groups.append(jnp.stack([a, b], axis=1).reshape(32, x.shape[1]))
    o_ref[...] = jnp.concatenate(groups, axis=0).T.astype(jnp.bfloat16)


def _features_to_table(features):
    blk = 2048
    return pl.pallas_call(
        _transpose_body,
        grid=(N // blk,),
        in_specs=[pl.BlockSpec((B, blk), lambda i: (0, i))],
        out_specs=pl.BlockSpec((blk, B), lambda i: (i, 0)),
        out_shape=jax.ShapeDtypeStruct((N, B), jnp.bfloat16),
    )(features)


def _combine_body(p_ref, k_ref, b_ref, o_ref):
    r = p_ref[0] * k_ref[0] + p_ref[1] * k_ref[1] + b_ref[0]
    o_ref[...] = jnp.clip(r, 0.0, 1.0).T


def _combine(partials, kern, bias):
    blk = 4096
    return pl.pallas_call(
        _combine_body,
        grid=(N // blk,),
        in_specs=[
            pl.BlockSpec((2, blk, B), lambda i: (0, i, 0)),
            pl.BlockSpec(memory_space=pltpu.SMEM),
            pl.BlockSpec(memory_space=pltpu.SMEM),
        ],
        out_specs=pl.BlockSpec((B, blk), lambda i: (0, i)),
        out_shape=jax.ShapeDtypeStruct((B, N), jnp.float32),
    )(partials, kern, bias)


def _sc_body(ft_hbm, cols_hbm, rows_hbm, vals_hbm, out_hbm,
             cols_c, rows_c, vals_c, gbuf, sbuf, acc,
             csems, gsems, ssems):
    c = lax.axis_index("c")
    s = lax.axis_index("s")

    def stage_slice(g):
        return pl.ds(c * (NNZ // G) + s * BLOCKS + g * GRP, GRP)

    def stage_start(g, q):
        sl = stage_slice(g)
        pltpu.async_copy(cols_hbm.at[sl], cols_c.at[q], csems[q])
        pltpu.async_copy(rows_hbm.at[sl], rows_c.at[q], csems[q])
        pltpu.async_copy(vals_hbm.at[sl], vals_c.at[q], csems[q])

    def stage_wait(g, q):
        sl = stage_slice(g)
        pltpu.make_async_copy(cols_hbm.at[sl], cols_c.at[q], csems[q]).wait()
        pltpu.make_async_copy(rows_hbm.at[sl], rows_c.at[q], csems[q]).wait()
        pltpu.make_async_copy(vals_hbm.at[sl], vals_c.at[q], csems[q]).wait()

    def gather(q, jj, gj):
        return pltpu.make_async_copy(ft_hbm.at[cols_c.at[q, jj]],
                                     gbuf.at[gj], gsems[gj])

    def scatter(q, jj, sj):
        return pltpu.make_async_copy(sbuf.at[sj], acc.at[rows_c.at[q, jj]],
                                     ssems[sj])

    def compute(q, jj, gj, sj):
        @plsc.parallel_loop(0, G, unroll=8)
        def _(e):
            val = plsc.load_gather(
                vals_c.at[q, jj], [jnp.full((LANES,), e, jnp.int32)])
            for h in range(B // (2 * LANES)):
                gv = gbuf[gj, e, pl.ds(2 * LANES * h, 2 * LANES)]
                a, b = plsc.unpack(gv, format=plsc.PackFormat.INTERLEAVED)
                sbuf[sj, e, pl.ds(2 * LANES * h, LANES)] = a * val
                sbuf[sj, e, pl.ds(2 * LANES * h + LANES, LANES)] = b * val

    for q in range(min(3, CRING - 1)):
        stage_start(q, q)

    zero = jnp.zeros((LANES,), jnp.float32)

    @pl.loop(0, G)
    def _(i):
        for k in range(B // LANES):
            sbuf[0, i, pl.ds(k * LANES, LANES)] = zero

    for j in range(ROWS_PER_TILE // G):
        pltpu.sync_copy(sbuf.at[0], acc.at[pl.ds(s * ROWS_PER_TILE + j * G, G)])

    plsc.subcore_barrier()

    stage_wait(0, 0)
    stage_wait(1, 1)
    for jj in range(GRING):
        gather(0, jj, jj).start()

    @pl.loop(0, NGRP // CRING)
    def _(m):
        for q in range(CRING):
            g = m * CRING + q
            for jj in range(GRP):
                bb = g * GRP + jj
                gj = jj % GRING
                sj = jj % SRING

                if jj == 0:
                    @pl.when(jnp.logical_and(g + 1 < NGRP, g > 0))
                    def _():
                        stage_wait(g + 1, (q + 1) % CRING)

                if jj == 2:
                    @pl.when(g + 3 < NGRP)
                    def _():
                        stage_start(g + 3, (q + 3) % CRING)

                gather(q, jj, gj).wait()

                if jj >= SRING:
                    scatter(q, jj - SRING, sj).wait()
                else:
                    @pl.when(g > 0)
                    def _():
                        scatter((q + CRING - 1) % CRING, GRP - SRING + jj,
                                sj).wait()

                compute(q, jj, gj, sj)

                @pl.when(bb + GRING < BLOCKS)
                def _():
                    gather((q + 1) % CRING, jj, gj).start()

                scatter(q, jj, sj).start(add=True)

    for u in range(SRING):
        scatter(CRING - 1, GRP - SRING + u, u % SRING).wait()

    plsc.subcore_barrier()

    for j in range(ROWS_PER_TILE // G):
        sl = pl.ds(s * ROWS_PER_TILE + j * G, G)
        pltpu.sync_copy(acc.at[sl], out_hbm.at[c].at[sl])


def _spmm_sc(ft, cols, rows, vals):
    mesh = plsc.VectorSubcoreMesh(
        core_axis_name="c", subcore_axis_name="s",
        num_cores=NUM_CORES, num_subcores=NUM_SUBCORES)
    cp = pltpu.CompilerParams()
    if "needs_layout_passes" in pltpu.CompilerParams.__dataclass_fields__:
        cp = dataclasses.replace(cp, needs_layout_passes=False)
    if "use_tc_tiling_on_sc" in pltpu.CompilerParams.__dataclass_fields__:
        cp = dataclasses.replace(cp, use_tc_tiling_on_sc=False)
    fn = pl.kernel(
        _sc_body,
        out_type=jax.ShapeDtypeStruct((2, N, B), jnp.float32),
        mesh=mesh,
        compiler_params=cp,
        scratch_types=[
            pltpu.VMEM((CRING, GRP, G), jnp.int32),
            pltpu.VMEM((CRING, GRP, G), jnp.int32),
            pltpu.VMEM((CRING, GRP, G), jnp.float32),
            pltpu.VMEM((GRING, G, B), jnp.bfloat16),
            pltpu.VMEM((SRING, G, B), jnp.float32),
            pltpu.VMEM_SHARED((N, B), jnp.float32),
            [pltpu.SemaphoreType.DMA] * CRING,
            [pltpu.SemaphoreType.DMA] * GRING,
            [pltpu.SemaphoreType.DMA] * SRING,
        ],
    )
    return fn(ft, cols, rows, vals)


def kernel(features, vals0, vals1, kernel, bias, rows0, cols0, rows1, cols1):
    ft = _features_to_table(features)
    shp = (2 * NNZ // G, G)
    partials = _spmm_sc(ft,
                        jnp.concatenate([cols0, cols1]).reshape(shp),
                        jnp.concatenate([rows0, rows1]).reshape(shp),
                        jnp.concatenate([vals0, vals1]).reshape(shp))
    return _combine(partials, kernel.reshape(2), bias)
